# baseline (speedup 1.0000x reference)
import functools

import jax
import jax.numpy as jnp
from jax import lax
from jax.experimental import pallas as pl
from jax.experimental.pallas import tpu as pltpu
from jax.experimental.pallas import tpu_sc as plsc

P = 14
NH = 12
NE = 7
B = 4
S = 256
T = B * S
D = 768
FF = 3072
DH = D // NH
KPAD = 640
NLANE = 128
BT = 128
MB = 24
PTOT = MB * BT


def _ln(x, g, b):
    m = jnp.mean(x, axis=-1, keepdims=True)
    v = jnp.mean((x - m) * (x - m), axis=-1, keepdims=True)
    return (x - m) * jax.lax.rsqrt(v + 1e-5) * g + b


def _enc_body(pat_ref, pw_ref, pb_ref, pos_ref, g1_ref, b1_ref,
              wq_ref, bq_ref, wk_ref, bk_ref, wv_ref, bv_ref,
              wo_ref, bo_ref, h_ref, attn_ref):
    pat = pat_ref[0]
    h0 = jnp.dot(pat, pw_ref[...], preferred_element_type=jnp.float32)
    h0 = h0 + pb_ref[...] + pos_ref[...]
    xl = _ln(h0, g1_ref[...], b1_ref[...])
    q = jnp.dot(xl, wq_ref[...], preferred_element_type=jnp.float32) + bq_ref[...]
    k = jnp.dot(xl, wk_ref[...], preferred_element_type=jnp.float32) + bk_ref[...]
    v = jnp.dot(xl, wv_ref[...], preferred_element_type=jnp.float32) + bv_ref[...]
    scale = 1.0 / (DH ** 0.5)
    outs = []
    for hh in range(NH):
        sl = slice(hh * DH, (hh + 1) * DH)
        qh, kh, vh = q[:, sl], k[:, sl], v[:, sl]
        sc = jax.lax.dot_general(qh, kh, (((1,), (1,)), ((), ())),
                                 preferred_element_type=jnp.float32) * scale
        mx = jnp.max(sc, axis=-1, keepdims=True)
        e = jnp.exp(sc - mx)
        pr = e / jnp.sum(e, axis=-1, keepdims=True)
        attn_ref[0, hh] = pr
        outs.append(jnp.dot(pr, vh, preferred_element_type=jnp.float32))
    sa = jnp.concatenate(outs, axis=1)
    sa = jnp.dot(sa, wo_ref[...], preferred_element_type=jnp.float32) + bo_ref[...]
    h_ref[0] = h0 + sa


def _encoder(patches, pw, pb, pos, g1, b1, wq, bq, wk, bk, wv, bv, wo, bo):
    full = lambda shp: pl.BlockSpec(shp, lambda i: (0,) * len(shp))
    return pl.pallas_call(
        _enc_body,
        grid=(B,),
        in_specs=[
            pl.BlockSpec((1, S, KPAD), lambda i: (i, 0, 0)),
            full((KPAD, D)), full((1, D)), full((S, D)),
            full((1, D)), full((1, D)),
            full((D, D)), full((1, D)), full((D, D)), full((1, D)),
            full((D, D)), full((1, D)), full((D, D)), full((1, D)),
        ],
        out_specs=[
            pl.BlockSpec((1, S, D), lambda i: (i, 0, 0)),
            pl.BlockSpec((1, NH, S, S), lambda i: (i, 0, 0, 0)),
        ],
        out_shape=[
            jax.ShapeDtypeStruct((B, S, D), jnp.float32),
            jax.ShapeDtypeStruct((B, NH, S, S), jnp.float32),
        ],
    )(patches, pw, pb, pos, g1, b1, wq, bq, wk, bk, wv, bv, wo, bo)


def _router_body(h_ref, g2_ref, b2_ref, wr_ref, br_ref,
                 masks_ref, loss_ref, pos_ref, w_ref):
    h = h_ref[...]
    rin = _ln(h, g2_ref[...], b2_ref[...])
    logits = jnp.dot(rin, wr_ref[...], preferred_element_type=jnp.float32) + br_ref[...]
    lane = jax.lax.broadcasted_iota(jnp.int32, (T, NLANE), 1)
    logits = jnp.where(lane < NE, logits, jnp.float32(-1e30))
    mx = jnp.max(logits, axis=-1, keepdims=True)
    e = jnp.exp(logits - mx)
    probs = e / jnp.sum(e, axis=-1, keepdims=True)
    v1 = jnp.max(probs, axis=-1, keepdims=True)
    i1 = jnp.argmax(probs, axis=-1).reshape(T, 1)
    p2 = jnp.where(lane == i1, -1.0, probs)
    v2 = jnp.max(p2, axis=-1, keepdims=True)
    i2 = jnp.argmax(p2, axis=-1).reshape(T, 1)
    wsum = v1 + v2
    w1 = v1 / wsum
    w2 = v2 / wsum
    m128 = jnp.where(lane == i1, w1, 0.0) + jnp.where(lane == i2, w2, 0.0)
    masks_ref[...] = m128[:, :NE]
    imp = jnp.mean(probs, axis=0, keepdims=True)
    load = jnp.mean((m128 > 0).astype(jnp.float32), axis=0, keepdims=True)
    loss = jnp.float32(NE) * jnp.sum(imp * load, axis=-1, keepdims=True)
    loss_ref[...] = jnp.broadcast_to(loss, (1, NLANE))

    ind1 = (lane == i1).astype(jnp.float32)
    ind2 = (lane == i2).astype(jnp.float32)
    r_io = jax.lax.broadcasted_iota(jnp.int32, (T, T), 0)
    c_io = jax.lax.broadcasted_iota(jnp.int32, (T, T), 1)
    ltri = (c_io < r_io).astype(jnp.float32)
    rank1 = jnp.sum(jnp.dot(ltri, ind1, preferred_element_type=jnp.float32) * ind1,
                    axis=-1, keepdims=True)
    rank2 = jnp.sum(jnp.dot(ltri, ind2, preferred_element_type=jnp.float32) * ind2,
                    axis=-1, keepdims=True)
    cnt1 = jnp.sum(ind1, axis=0, keepdims=True)
    cnt = cnt1 + jnp.sum(ind2, axis=0, keepdims=True)
    nblk = (cnt.astype(jnp.int32) + (BT - 1)) // BT
    pcnt = (nblk * BT).astype(jnp.float32)
    ur_io = jax.lax.broadcasted_iota(jnp.int32, (NLANE, NLANE), 0)
    uc_io = jax.lax.broadcasted_iota(jnp.int32, (NLANE, NLANE), 1)
    utri = (ur_io < uc_io).astype(jnp.float32)
    poff = jnp.dot(pcnt, utri, preferred_element_type=jnp.float32)
    pos1 = jnp.sum(poff * ind1, axis=-1, keepdims=True) + rank1
    pos2 = jnp.sum((poff + cnt1) * ind2, axis=-1, keepdims=True) + rank2
    posf = jnp.where(lane == 0, pos1, jnp.where(lane == 1, pos2, 0.0))
    pos_ref[...] = posf.astype(jnp.int32)
    w_ref[...] = jnp.where(lane == 0, w1, jnp.where(lane == 1, w2, 0.0))


def _router(h2d, g2, b2, wr, br):
    full = lambda shp: pl.BlockSpec(shp, lambda: (0,) * len(shp))
    return pl.pallas_call(
        _router_body,
        in_specs=[full((T, D)), full((1, D)), full((1, D)),
                  full((D, NLANE)), full((1, NLANE))],
        out_specs=[full((T, NE)), full((1, NLANE)),
                   full((T, NLANE)), full((T, NLANE))],
        out_shape=[
            jax.ShapeDtypeStruct((T, NE), jnp.float32),
            jax.ShapeDtypeStruct((1, NLANE), jnp.float32),
            jax.ShapeDtypeStruct((T, NLANE), jnp.int32),
            jax.ShapeDtypeStruct((T, NLANE), jnp.float32),
        ],
    )(h2d, g2, b2, wr, br)


def _inv_body(pc_ref, tcol_ref, tok_ref):
    i = pl.program_id(0)
    q_io = jax.lax.broadcasted_iota(jnp.int32, (PTOT // 3, 2 * T), 0) + i * (PTOT // 3)
    eq = (q_io == pc_ref[...]).astype(jnp.float32)
    tok = jnp.dot(eq, tcol_ref[...], preferred_element_type=jnp.float32,
                  precision=jax.lax.Precision.HIGHEST)
    tok_ref[...] = tok.astype(jnp.int32)


def _invert(poscat_row, tcol):
    return pl.pallas_call(
        _inv_body,
        grid=(3,),
        in_specs=[pl.BlockSpec((1, 2 * T), lambda i: (0, 0)),
                  pl.BlockSpec((2 * T, NLANE), lambda i: (0, 0))],
        out_specs=pl.BlockSpec((PTOT // 3, NLANE), lambda i: (i, 0)),
        out_shape=jax.ShapeDtypeStruct((PTOT, NLANE), jnp.int32),
    )(poscat_row, tcol)


def _sc_dispatch(h2d, tok_sorted):
    info = plsc.get_sparse_core_info()
    nc, ns = info.num_cores, info.num_subcores
    nw = nc * ns
    rpw = PTOT // nw
    mesh = plsc.VectorSubcoreMesh(core_axis_name="c", subcore_axis_name="s")

    @functools.partial(
        pl.kernel, mesh=mesh,
        compiler_params=pltpu.CompilerParams(needs_layout_passes=False),
        out_type=jax.ShapeDtypeStruct((PTOT, D), jnp.float32),
        scratch_types=[
            pltpu.VMEM((rpw,), jnp.int32),
            pltpu.VMEM((rpw, D), jnp.float32),
            pltpu.SemaphoreType.DMA,
        ],
    )
    def k(h_hbm, tok_hbm, x_hbm, idx_v, rows_v, sem):
        wid = lax.axis_index("s") * nc + lax.axis_index("c")
        base = wid * rpw
        pltpu.sync_copy(tok_hbm.at[pl.ds(base, rpw)], idx_v)
        pltpu.async_copy(h_hbm.at[idx_v], rows_v, sem).wait()
        pltpu.sync_copy(rows_v, x_hbm.at[pl.ds(base, rpw)])

    return k(h2d, tok_sorted)


def _sc_combine(eo_sorted, pos1, pos2):
    info = plsc.get_sparse_core_info()
    nc, ns = info.num_cores, info.num_subcores
    nw = nc * ns
    tpw = T // nw
    mesh = plsc.VectorSubcoreMesh(core_axis_name="c", subcore_axis_name="s")

    @functools.partial(
        pl.kernel, mesh=mesh,
        out_type=[jax.ShapeDtypeStruct((T, D), jnp.float32),
                  jax.ShapeDtypeStruct((T, D), jnp.float32)],
        scratch_types=[
            pltpu.VMEM((tpw,), jnp.int32),
            pltpu.VMEM((tpw, D), jnp.float32),
            pltpu.SemaphoreType.DMA,
        ],
    )
    def k(eo_hbm, p1_hbm, p2_hbm, o1_hbm, o2_hbm, idx_v, rows_v, sem):
        wid = lax.axis_index("s") * nc + lax.axis_index("c")
        base = wid * tpw
        pltpu.sync_copy(p1_hbm.at[pl.ds(base, tpw)], idx_v)
        pltpu.async_copy(eo_hbm.at[idx_v], rows_v, sem).wait()
        pltpu.sync_copy(rows_v, o1_hbm.at[pl.ds(base, tpw)])
        pltpu.sync_copy(p2_hbm.at[pl.ds(base, tpw)], idx_v)
        pltpu.async_copy(eo_hbm.at[idx_v], rows_v, sem).wait()
        pltpu.sync_copy(rows_v, o2_hbm.at[pl.ds(base, tpw)])

    return k(eo_sorted, pos1, pos2)


def _gexp_body(be_ref, x_ref, w1_ref, b1_ref, w2_ref, b2_ref, out_ref):
    i = pl.program_id(0)

    @pl.when(be_ref[i] < NE)
    def _():
        h1 = jnp.dot(x_ref[...], w1_ref[0], preferred_element_type=jnp.float32)
        g = jax.nn.gelu(h1 + b1_ref[0])
        out_ref[...] = (jnp.dot(g, w2_ref[0], preferred_element_type=jnp.float32)
                        + b2_ref[0])


def _gexperts(be, x_sorted, ew1, eb1, ew2, eb2):
    def emap(i, be_ref):
        return (jnp.minimum(be_ref[i], NE - 1), 0, 0)
    spec = pltpu.PrefetchScalarGridSpec(
        num_scalar_prefetch=1,
        grid=(MB,),
        in_specs=[
            pl.BlockSpec((BT, D), lambda i, be_ref: (i, 0)),
            pl.BlockSpec((1, D, FF), emap),
            pl.BlockSpec((1, 1, FF), emap),
            pl.BlockSpec((1, FF, D), emap),
            pl.BlockSpec((1, 1, D), emap),
        ],
        out_specs=pl.BlockSpec((BT, D), lambda i, be_ref: (i, 0)),
    )
    return pl.pallas_call(
        _gexp_body,
        grid_spec=spec,
        out_shape=jax.ShapeDtypeStruct((PTOT, D), jnp.float32),
    )(be, x_sorted, ew1, eb1, ew2, eb2)


def _fin_body(eo1_ref, eo2_ref, w_ref, g3_ref, b3_ref, cw_ref, cb_ref,
              fv_ref, cls_ref):
    w1 = w_ref[...][:, 0:1]
    w2 = w_ref[...][:, 1:2]
    acc = w1 * eo1_ref[...] + w2 * eo2_ref[...]
    o = _ln(acc, g3_ref[...], b3_ref[...])
    rows = [jnp.mean(o[b * S:(b + 1) * S], axis=0, keepdims=True)
            for b in range(B)]
    fv = jnp.concatenate(rows, axis=0)
    fv_ref[...] = fv
    cls_ref[...] = jnp.dot(fv, cw_ref[...], preferred_element_type=jnp.float32) + cb_ref[...]


def _final(eo1, eo2, wpair, g3, b3, cw, cb):
    full = lambda shp: pl.BlockSpec(shp, lambda: (0,) * len(shp))
    return pl.pallas_call(
        _fin_body,
        in_specs=[full((T, D)), full((T, D)), full((T, NLANE)),
                  full((1, D)), full((1, D)), full((D, D)), full((1, D))],
        out_specs=[full((B, D)), full((B, D))],
        out_shape=[jax.ShapeDtypeStruct((B, D), jnp.float32),
                   jax.ShapeDtypeStruct((B, D), jnp.float32)],
    )(eo1, eo2, wpair, g3, b3, cw, cb)


def kernel(x, params):
    p = params
    hp = wp = 224 // P
    patches = x.reshape(B, 3, hp, P, wp, P).transpose(0, 2, 4, 3, 5, 1)
    patches = patches.reshape(B, S, P * P * 3)
    patches = jnp.pad(patches, ((0, 0), (0, 0), (0, KPAD - P * P * 3)))
    pw = jnp.pad(p['patch_W'], ((0, KPAD - P * P * 3), (0, 0)))
    row = lambda a: a.reshape(1, -1)
    pos = p['pos'].reshape(S, D)

    h, attn = _encoder(patches, pw, row(p['patch_b']), pos,
                       row(p['ln1_g']), row(p['ln1_b']),
                       p['Wq'], row(p['bq']), p['Wk'], row(p['bk']),
                       p['Wv'], row(p['bv']), p['Wo'], row(p['bo']))

    h2d = h.reshape(T, D)
    wr = jnp.pad(p['router_W'], ((0, 0), (0, NLANE - NE)))
    br = jnp.pad(p['router_b'], ((0, NLANE - NE))).reshape(1, NLANE)
    masks2d, loss_row, posout, wout = _router(
        h2d, row(p['ln2_g']), row(p['ln2_b']), wr, br)

    pos1 = posout[:, 0]
    pos2 = posout[:, 1]
    cnt = jnp.sum(masks2d > 0, axis=0).astype(jnp.int32)
    nblk = (cnt + (BT - 1)) // BT
    cumblk = jnp.cumsum(nblk)
    be = jnp.sum((jnp.arange(MB, dtype=jnp.int32)[:, None]
                  >= cumblk[None, :]).astype(jnp.int32), axis=1)

    poscat = jnp.concatenate([pos1, pos2], axis=0)
    tvals = (jnp.arange(2 * T, dtype=jnp.float32) % T).reshape(2 * T, 1)
    tcol = jnp.pad(tvals, ((0, 0), (0, NLANE - 1)))
    tok_sorted = _invert(poscat.reshape(1, 2 * T), tcol)[:, 0]
    x_sorted = _sc_dispatch(h2d, tok_sorted)
    eo_sorted = _gexperts(be, x_sorted,
                          p['exp_W1'], p['exp_b1'].reshape(NE, 1, FF),
                          p['exp_W2'], p['exp_b2'].reshape(NE, 1, D))
    eo1, eo2 = _sc_combine(eo_sorted, pos1, pos2)

    fv, cls = _final(eo1, eo2, wout, row(p['ln3_g']), row(p['ln3_b']),
                     p['cls_W'], row(p['cls_b']))

    masks = masks2d.reshape(B, S, NE)
    router_loss = loss_row[0, 0]
    return fv, cls, router_loss, masks, attn


# speedup vs baseline: 1.3813x; 1.3813x over previous
import functools

import jax
import jax.numpy as jnp
from jax import lax
from jax.experimental import pallas as pl
from jax.experimental.pallas import tpu as pltpu
from jax.experimental.pallas import tpu_sc as plsc

P = 14
NH = 12
NE = 7
B = 4
S = 256
T = B * S
D = 768
FF = 3072
DH = D // NH
KPAD = 640
NLANE = 128
BT = 128
MB = 24
PTOT = MB * BT


def _ln(x, g, b):
    m = jnp.mean(x, axis=-1, keepdims=True)
    v = jnp.mean((x - m) * (x - m), axis=-1, keepdims=True)
    return (x - m) * jax.lax.rsqrt(v + 1e-5) * g + b


def _enc_body(pat_ref, pw_ref, pb_ref, pos_ref, g1_ref, b1_ref,
              wq_ref, bq_ref, wk_ref, bk_ref, wv_ref, bv_ref,
              wo_ref, bo_ref, h_ref, attn_ref):
    pat = pat_ref[0]
    h0 = jnp.dot(pat, pw_ref[...], preferred_element_type=jnp.float32)
    h0 = h0 + pb_ref[...] + pos_ref[...]
    xl = _ln(h0, g1_ref[...], b1_ref[...])
    q = jnp.dot(xl, wq_ref[...], preferred_element_type=jnp.float32) + bq_ref[...]
    k = jnp.dot(xl, wk_ref[...], preferred_element_type=jnp.float32) + bk_ref[...]
    v = jnp.dot(xl, wv_ref[...], preferred_element_type=jnp.float32) + bv_ref[...]
    scale = 1.0 / (DH ** 0.5)
    outs = []
    for hh in range(NH):
        sl = slice(hh * DH, (hh + 1) * DH)
        qh, kh, vh = q[:, sl], k[:, sl], v[:, sl]
        sc = jax.lax.dot_general(qh, kh, (((1,), (1,)), ((), ())),
                                 preferred_element_type=jnp.float32) * scale
        mx = jnp.max(sc, axis=-1, keepdims=True)
        e = jnp.exp(sc - mx)
        pr = e / jnp.sum(e, axis=-1, keepdims=True)
        attn_ref[0, hh] = pr
        outs.append(jnp.dot(pr, vh, preferred_element_type=jnp.float32))
    sa = jnp.concatenate(outs, axis=1)
    sa = jnp.dot(sa, wo_ref[...], preferred_element_type=jnp.float32) + bo_ref[...]
    h_ref[0] = h0 + sa


def _encoder(patches, pw, pb, pos, g1, b1, wq, bq, wk, bk, wv, bv, wo, bo):
    full = lambda shp: pl.BlockSpec(shp, lambda i: (0,) * len(shp))
    return pl.pallas_call(
        _enc_body,
        grid=(B,),
        in_specs=[
            pl.BlockSpec((1, S, KPAD), lambda i: (i, 0, 0)),
            full((KPAD, D)), full((1, D)), full((S, D)),
            full((1, D)), full((1, D)),
            full((D, D)), full((1, D)), full((D, D)), full((1, D)),
            full((D, D)), full((1, D)), full((D, D)), full((1, D)),
        ],
        out_specs=[
            pl.BlockSpec((1, S, D), lambda i: (i, 0, 0)),
            pl.BlockSpec((1, NH, S, S), lambda i: (i, 0, 0, 0)),
        ],
        out_shape=[
            jax.ShapeDtypeStruct((B, S, D), jnp.float32),
            jax.ShapeDtypeStruct((B, NH, S, S), jnp.float32),
        ],
    )(patches, pw, pb, pos, g1, b1, wq, bq, wk, bk, wv, bv, wo, bo)


def _router_body(h_ref, g2_ref, b2_ref, wr_ref, br_ref,
                 masks_ref, loss_ref, pos_ref, w_ref):
    h = h_ref[...]
    rin = _ln(h, g2_ref[...], b2_ref[...])
    logits = jnp.dot(rin, wr_ref[...], preferred_element_type=jnp.float32) + br_ref[...]
    lane = jax.lax.broadcasted_iota(jnp.int32, (T, NLANE), 1)
    logits = jnp.where(lane < NE, logits, jnp.float32(-1e30))
    mx = jnp.max(logits, axis=-1, keepdims=True)
    e = jnp.exp(logits - mx)
    probs = e / jnp.sum(e, axis=-1, keepdims=True)
    v1 = jnp.max(probs, axis=-1, keepdims=True)
    i1 = jnp.argmax(probs, axis=-1).reshape(T, 1)
    p2 = jnp.where(lane == i1, -1.0, probs)
    v2 = jnp.max(p2, axis=-1, keepdims=True)
    i2 = jnp.argmax(p2, axis=-1).reshape(T, 1)
    wsum = v1 + v2
    w1 = v1 / wsum
    w2 = v2 / wsum
    m128 = jnp.where(lane == i1, w1, 0.0) + jnp.where(lane == i2, w2, 0.0)
    masks_ref[...] = m128[:, :NE]
    imp = jnp.mean(probs, axis=0, keepdims=True)
    load = jnp.mean((m128 > 0).astype(jnp.float32), axis=0, keepdims=True)
    loss = jnp.float32(NE) * jnp.sum(imp * load, axis=-1, keepdims=True)
    loss_ref[...] = jnp.broadcast_to(loss, (1, NLANE))

    ind1 = (lane == i1).astype(jnp.float32)
    ind2 = (lane == i2).astype(jnp.float32)
    r_io = jax.lax.broadcasted_iota(jnp.int32, (T, T), 0)
    c_io = jax.lax.broadcasted_iota(jnp.int32, (T, T), 1)
    ltri = (c_io < r_io).astype(jnp.float32)
    rank1 = jnp.sum(jnp.dot(ltri, ind1, preferred_element_type=jnp.float32) * ind1,
                    axis=-1, keepdims=True)
    rank2 = jnp.sum(jnp.dot(ltri, ind2, preferred_element_type=jnp.float32) * ind2,
                    axis=-1, keepdims=True)
    cnt1 = jnp.sum(ind1, axis=0, keepdims=True)
    cnt = cnt1 + jnp.sum(ind2, axis=0, keepdims=True)
    nblk = (cnt.astype(jnp.int32) + (BT - 1)) // BT
    pcnt = (nblk * BT).astype(jnp.float32)
    ur_io = jax.lax.broadcasted_iota(jnp.int32, (NLANE, NLANE), 0)
    uc_io = jax.lax.broadcasted_iota(jnp.int32, (NLANE, NLANE), 1)
    utri = (ur_io < uc_io).astype(jnp.float32)
    poff = jnp.dot(pcnt, utri, preferred_element_type=jnp.float32)
    pos1 = jnp.sum(poff * ind1, axis=-1, keepdims=True) + rank1
    pos2 = jnp.sum((poff + cnt1) * ind2, axis=-1, keepdims=True) + rank2
    posf = jnp.where(lane == 0, pos1, jnp.where(lane == 1, pos2, 0.0))
    pos_ref[...] = posf.astype(jnp.int32)
    w_ref[...] = jnp.where(lane == 0, w1, jnp.where(lane == 1, w2, 0.0))


def _router(h2d, g2, b2, wr, br):
    full = lambda shp: pl.BlockSpec(shp, lambda: (0,) * len(shp))
    return pl.pallas_call(
        _router_body,
        in_specs=[full((T, D)), full((1, D)), full((1, D)),
                  full((D, NLANE)), full((1, NLANE))],
        out_specs=[full((T, NE)), full((1, NLANE)),
                   full((T, NLANE)), full((T, NLANE))],
        out_shape=[
            jax.ShapeDtypeStruct((T, NE), jnp.float32),
            jax.ShapeDtypeStruct((1, NLANE), jnp.float32),
            jax.ShapeDtypeStruct((T, NLANE), jnp.int32),
            jax.ShapeDtypeStruct((T, NLANE), jnp.float32),
        ],
    )(h2d, g2, b2, wr, br)


def _sc_combine(eo_sorted, pos1, pos2):
    info = plsc.get_sparse_core_info()
    nc, ns = info.num_cores, info.num_subcores
    nw = nc * ns
    tpw = T // nw
    mesh = plsc.VectorSubcoreMesh(core_axis_name="c", subcore_axis_name="s")

    @functools.partial(
        pl.kernel, mesh=mesh,
        out_type=[jax.ShapeDtypeStruct((T, D), jnp.float32),
                  jax.ShapeDtypeStruct((T, D), jnp.float32)],
        scratch_types=[
            pltpu.VMEM((tpw,), jnp.int32),
            pltpu.VMEM((tpw, D), jnp.float32),
            pltpu.SemaphoreType.DMA,
        ],
    )
    def k(eo_hbm, p1_hbm, p2_hbm, o1_hbm, o2_hbm, idx_v, rows_v, sem):
        wid = lax.axis_index("s") * nc + lax.axis_index("c")
        base = wid * tpw
        pltpu.sync_copy(p1_hbm.at[pl.ds(base, tpw)], idx_v)
        pltpu.async_copy(eo_hbm.at[idx_v], rows_v, sem).wait()
        pltpu.sync_copy(rows_v, o1_hbm.at[pl.ds(base, tpw)])
        pltpu.sync_copy(p2_hbm.at[pl.ds(base, tpw)], idx_v)
        pltpu.async_copy(eo_hbm.at[idx_v], rows_v, sem).wait()
        pltpu.sync_copy(rows_v, o2_hbm.at[pl.ds(base, tpw)])

    return k(eo_sorted, pos1, pos2)


def _gexp_body(be_ref, h_ref, p1_ref, p2_ref, w1_ref, b1_ref, w2_ref, b2_ref,
               out_ref):
    i = pl.program_id(0)

    @pl.when(be_ref[i] < NE)
    def _():
        qcol = jax.lax.broadcasted_iota(jnp.int32, (BT, T), 0) + i * BT
        eq = jnp.logical_or(p1_ref[...] == qcol, p2_ref[...] == qcol)
        x = jnp.dot(eq.astype(jnp.float32), h_ref[...],
                    preferred_element_type=jnp.float32)
        h1 = jnp.dot(x, w1_ref[0], preferred_element_type=jnp.float32)
        g = jax.nn.gelu(h1 + b1_ref[0])
        out_ref[...] = (jnp.dot(g, w2_ref[0], preferred_element_type=jnp.float32)
                        + b2_ref[0])


def _gexperts(be, h2d, p1row, p2row, ew1, eb1, ew2, eb2):
    def emap(i, be_ref):
        return (jnp.minimum(be_ref[i], NE - 1), 0, 0)
    spec = pltpu.PrefetchScalarGridSpec(
        num_scalar_prefetch=1,
        grid=(MB,),
        in_specs=[
            pl.BlockSpec((T, D), lambda i, be_ref: (0, 0)),
            pl.BlockSpec((1, T), lambda i, be_ref: (0, 0)),
            pl.BlockSpec((1, T), lambda i, be_ref: (0, 0)),
            pl.BlockSpec((1, D, FF), emap),
            pl.BlockSpec((1, 1, FF), emap),
            pl.BlockSpec((1, FF, D), emap),
            pl.BlockSpec((1, 1, D), emap),
        ],
        out_specs=pl.BlockSpec((BT, D), lambda i, be_ref: (i, 0)),
    )
    return pl.pallas_call(
        _gexp_body,
        grid_spec=spec,
        out_shape=jax.ShapeDtypeStruct((PTOT, D), jnp.float32),
    )(be, h2d, p1row, p2row, ew1, eb1, ew2, eb2)


def _fin_body(eo1_ref, eo2_ref, w_ref, g3_ref, b3_ref, cw_ref, cb_ref,
              fv_ref, cls_ref):
    w1 = w_ref[...][:, 0:1]
    w2 = w_ref[...][:, 1:2]
    acc = w1 * eo1_ref[...] + w2 * eo2_ref[...]
    o = _ln(acc, g3_ref[...], b3_ref[...])
    rows = [jnp.mean(o[b * S:(b + 1) * S], axis=0, keepdims=True)
            for b in range(B)]
    fv = jnp.concatenate(rows, axis=0)
    fv_ref[...] = fv
    cls_ref[...] = jnp.dot(fv, cw_ref[...], preferred_element_type=jnp.float32) + cb_ref[...]


def _final(eo1, eo2, wpair, g3, b3, cw, cb):
    full = lambda shp: pl.BlockSpec(shp, lambda: (0,) * len(shp))
    return pl.pallas_call(
        _fin_body,
        in_specs=[full((T, D)), full((T, D)), full((T, NLANE)),
                  full((1, D)), full((1, D)), full((D, D)), full((1, D))],
        out_specs=[full((B, D)), full((B, D))],
        out_shape=[jax.ShapeDtypeStruct((B, D), jnp.float32),
                   jax.ShapeDtypeStruct((B, D), jnp.float32)],
    )(eo1, eo2, wpair, g3, b3, cw, cb)


def kernel(x, params):
    p = params
    hp = wp = 224 // P
    patches = x.reshape(B, 3, hp, P, wp, P).transpose(0, 2, 4, 3, 5, 1)
    patches = patches.reshape(B, S, P * P * 3)
    patches = jnp.pad(patches, ((0, 0), (0, 0), (0, KPAD - P * P * 3)))
    pw = jnp.pad(p['patch_W'], ((0, KPAD - P * P * 3), (0, 0)))
    row = lambda a: a.reshape(1, -1)
    pos = p['pos'].reshape(S, D)

    h, attn = _encoder(patches, pw, row(p['patch_b']), pos,
                       row(p['ln1_g']), row(p['ln1_b']),
                       p['Wq'], row(p['bq']), p['Wk'], row(p['bk']),
                       p['Wv'], row(p['bv']), p['Wo'], row(p['bo']))

    h2d = h.reshape(T, D)
    wr = jnp.pad(p['router_W'], ((0, 0), (0, NLANE - NE)))
    br = jnp.pad(p['router_b'], ((0, NLANE - NE))).reshape(1, NLANE)
    masks2d, loss_row, posout, wout = _router(
        h2d, row(p['ln2_g']), row(p['ln2_b']), wr, br)

    pos1 = posout[:, 0]
    pos2 = posout[:, 1]
    cnt = jnp.sum(masks2d > 0, axis=0).astype(jnp.int32)
    nblk = (cnt + (BT - 1)) // BT
    cumblk = jnp.cumsum(nblk)
    be = jnp.sum((jnp.arange(MB, dtype=jnp.int32)[:, None]
                  >= cumblk[None, :]).astype(jnp.int32), axis=1)

    eo_sorted = _gexperts(be, h2d, pos1.reshape(1, T), pos2.reshape(1, T),
                          p['exp_W1'], p['exp_b1'].reshape(NE, 1, FF),
                          p['exp_W2'], p['exp_b2'].reshape(NE, 1, D))
    eo1, eo2 = _sc_combine(eo_sorted, pos1, pos2)

    fv, cls = _final(eo1, eo2, wout, row(p['ln3_g']), row(p['ln3_b']),
                     p['cls_W'], row(p['cls_b']))

    masks = masks2d.reshape(B, S, NE)
    router_loss = loss_row[0, 0]
    return fv, cls, router_loss, masks, attn
